# baseline (speedup 1.0000x reference)
import functools

import jax
import jax.numpy as jnp
from jax import lax
from jax.experimental import pallas as pl
from jax.experimental.pallas import tpu as pltpu
from jax.experimental.pallas import tpu_sc as plsc

N = 10000
NP = 10240
E = 320000
C = 128
G = 64

NC = 2
NS = 16
NW = NC * NS
EPT = E // NW
K = 128
NCH = 79
EPP = NCH * K
PADE = EPP - EPT
RPT = NP // NS
RB = RPT // K
PH0 = 40

_mesh = plsc.VectorSubcoreMesh(core_axis_name="c", subcore_axis_name="s")


@functools.partial(
    pl.kernel,
    out_type=jax.ShapeDtypeStruct((NC, NP), jnp.float32),
    mesh=_mesh,
    scratch_types=[
        pltpu.VMEM((NCH, K), jnp.int32),
        pltpu.VMEM((K,), jnp.float32),
        pltpu.VMEM((RPT,), jnp.float32),
        pltpu.VMEM_SHARED((NP,), jnp.float32),
        pltpu.SemaphoreType.DMA,
    ],
)
def _deg_kernel(dst_hbm, out_hbm, idx_v, ones_v, stage_v, deg_sh, sem):
    c = lax.axis_index("c")
    s = lax.axis_index("s")
    wid = c * NS + s

    def fill_stage(i, _):
        stage_v[pl.ds(i * 16, 16)] = jnp.zeros((16,), jnp.float32)
        return 0
    lax.fori_loop(0, RPT // 16, fill_stage, 0)

    def fill_ones(i, _):
        ones_v[pl.ds(i * 16, 16)] = jnp.full((16,), 1.0, jnp.float32)
        return 0
    lax.fori_loop(0, K // 16, fill_ones, 0)

    pltpu.sync_copy(stage_v, deg_sh.at[pl.ds(s * RPT, RPT)])
    plsc.subcore_barrier()

    pltpu.sync_copy(dst_hbm.at[wid], idx_v)

    def body(j, _):
        pltpu.async_copy(ones_v, deg_sh.at[idx_v.at[j]], sem,
                         add=True).wait()
        return 0
    lax.fori_loop(0, NCH, body, 0)
    plsc.subcore_barrier()

    pltpu.sync_copy(deg_sh.at[pl.ds(s * RPT, RPT)], stage_v)
    pltpu.sync_copy(stage_v, out_hbm.at[c, pl.ds(s * RPT, RPT)])


@functools.partial(
    pl.kernel,
    out_type=jax.ShapeDtypeStruct((NC, NP, C), jnp.float32),
    mesh=_mesh,
    scratch_types=[
        pltpu.VMEM((PH0, K), jnp.int32),
        pltpu.VMEM((PH0, K), jnp.int32),
        pltpu.VMEM((2, K, C), jnp.float32),
        pltpu.VMEM_SHARED((NP, C), jnp.float32),
        pltpu.SemaphoreType.DMA,
        pltpu.SemaphoreType.DMA,
        pltpu.SemaphoreType.DMA,
        pltpu.SemaphoreType.DMA,
    ],
)
def _prop_kernel(hs_hbm, src_hbm, dst_hbm, out_hbm,
                 src_v, dst_v, rows_v, acc_sh,
                 gsem0, gsem1, ssem0, ssem1):
    c = lax.axis_index("c")
    s = lax.axis_index("s")
    wid = c * NS + s
    gsems = (gsem0, gsem1)
    ssems = (ssem0, ssem1)

    def fill_z(r, _):
        def fill_row(i, _):
            rows_v[0, r, pl.ds(i * 16, 16)] = jnp.zeros((16,), jnp.float32)
            return 0
        lax.fori_loop(0, C // 16, fill_row, 0)
        return 0
    lax.fori_loop(0, K, fill_z, 0)

    def zero_acc(r, _):
        pltpu.sync_copy(rows_v.at[0],
                        acc_sh.at[pl.ds(s * RPT + r * K, K)])
        return 0
    lax.fori_loop(0, RB, zero_acc, 0)
    plsc.subcore_barrier()

    def run_phase(base, nch):
        pltpu.sync_copy(src_hbm.at[wid, pl.ds(base, nch)],
                        src_v.at[pl.ds(0, nch)])
        pltpu.sync_copy(dst_hbm.at[wid, pl.ds(base, nch)],
                        dst_v.at[pl.ds(0, nch)])

        def gather(j, slot):
            return pltpu.async_copy(
                hs_hbm.at[src_v.at[j]], rows_v.at[slot], gsems[slot])

        gather(0, 0)

        def body(j, _):
            for r in range(2):
                ro = 1 - r

                @pl.when(lax.rem(j, 2) == r)
                def _():
                    @pl.when(j >= 1)
                    def _():
                        pltpu.make_async_copy(
                            rows_v.at[ro],
                            acc_sh.at[dst_v.at[j - 1]],
                            ssems[ro]).wait()

                    @pl.when(j + 1 < nch)
                    def _():
                        gather(j + 1, ro)
                    pltpu.make_async_copy(
                        hs_hbm.at[src_v.at[j]],
                        rows_v.at[r], gsems[r]).wait()
                    pltpu.async_copy(rows_v.at[r],
                                     acc_sh.at[dst_v.at[j]], ssems[r],
                                     add=True)
            return 0
        lax.fori_loop(0, nch, body, 0)
        t = nch - 1
        pltpu.make_async_copy(rows_v.at[t % 2],
                              acc_sh.at[dst_v.at[t]],
                              ssems[t % 2]).wait()

    run_phase(0, PH0)
    run_phase(PH0, NCH - PH0)
    plsc.subcore_barrier()

    def wb(r, _):
        slot = rows_v.at[0]
        pltpu.sync_copy(acc_sh.at[pl.ds(s * RPT + r * K, K)], slot)
        pltpu.sync_copy(slot, out_hbm.at[c, pl.ds(s * RPT + r * K, K)])
        return 0
    lax.fori_loop(0, RB, wb, 0)


BLK = 512


def _mm1_body(degp_ref, x_ref, w_ref, hs_ref, dinv_ref):
    deg = degp_ref[0] + degp_ref[1] + 1.0
    dinv = lax.rsqrt(deg)
    h = jnp.dot(x_ref[...], w_ref[...], preferred_element_type=jnp.float32)
    hs_ref[...] = dinv * h
    dinv_ref[...] = dinv


def _call_mm1(degp, x_p, W1):
    return pl.pallas_call(
        _mm1_body,
        grid=(NP // BLK,),
        in_specs=[
            pl.BlockSpec((NC, BLK, 1), lambda i: (0, i, 0)),
            pl.BlockSpec((BLK, C), lambda i: (i, 0)),
            pl.BlockSpec((C, C), lambda i: (0, 0)),
        ],
        out_specs=[
            pl.BlockSpec((BLK, C), lambda i: (i, 0)),
            pl.BlockSpec((BLK, 1), lambda i: (i, 0)),
        ],
        out_shape=[
            jax.ShapeDtypeStruct((NP, C), jnp.float32),
            jax.ShapeDtypeStruct((NP, 1), jnp.float32),
        ],
    )(degp, x_p, W1)


def _mid_body(accp_ref, hs_ref, dinv_ref, b_ref, w_ref, bat_ref,
              h1_ref, hs2_ref, sx_ref):
    i = pl.program_id(0)
    dinv = dinv_ref[...]
    pre = dinv * (accp_ref[0] + accp_ref[1] + hs_ref[...]) + b_ref[...]
    h1 = jnp.maximum(pre, 0.0)
    h1_ref[...] = h1
    hs2_ref[...] = dinv * jnp.dot(h1, w_ref[...],
                                  preferred_element_type=jnp.float32)
    oh = (bat_ref[...] == lax.broadcasted_iota(jnp.int32, (1, G), 1))
    oh = oh.astype(jnp.float32)
    part = lax.dot_general(oh, h1, (((0,), (0,)), ((), ())),
                           preferred_element_type=jnp.float32)

    @pl.when(i == 0)
    def _():
        sx_ref[...] = jnp.zeros_like(sx_ref)
    sx_ref[...] += part


def _call_mid(accp, hs1, dinv, b1, W2, bat):
    return pl.pallas_call(
        _mid_body,
        grid=(NP // BLK,),
        in_specs=[
            pl.BlockSpec((NC, BLK, C), lambda i: (0, i, 0)),
            pl.BlockSpec((BLK, C), lambda i: (i, 0)),
            pl.BlockSpec((BLK, 1), lambda i: (i, 0)),
            pl.BlockSpec((1, C), lambda i: (0, 0)),
            pl.BlockSpec((C, C), lambda i: (0, 0)),
            pl.BlockSpec((BLK, 1), lambda i: (i, 0)),
        ],
        out_specs=[
            pl.BlockSpec((BLK, C), lambda i: (i, 0)),
            pl.BlockSpec((BLK, C), lambda i: (i, 0)),
            pl.BlockSpec((G, C), lambda i: (0, 0)),
        ],
        out_shape=[
            jax.ShapeDtypeStruct((NP, C), jnp.float32),
            jax.ShapeDtypeStruct((NP, C), jnp.float32),
            jax.ShapeDtypeStruct((G, C), jnp.float32),
        ],
    )(accp, hs1, dinv, b1, W2, bat)


def _fin_body(accp_ref, hs_ref, dinv_ref, b_ref, bat_ref, h2_ref, sx_ref):
    i = pl.program_id(0)
    dinv = dinv_ref[...]
    pre = dinv * (accp_ref[0] + accp_ref[1] + hs_ref[...]) + b_ref[...]
    h2 = jnp.maximum(pre, 0.0)
    h2_ref[...] = h2
    oh = (bat_ref[...] == lax.broadcasted_iota(jnp.int32, (1, G), 1))
    oh = oh.astype(jnp.float32)
    part = lax.dot_general(oh, h2, (((0,), (0,)), ((), ())),
                           preferred_element_type=jnp.float32)

    @pl.when(i == 0)
    def _():
        sx_ref[...] = jnp.zeros_like(sx_ref)
    sx_ref[...] += part


def _call_fin(accp, hs2, dinv, b2, bat):
    return pl.pallas_call(
        _fin_body,
        grid=(NP // BLK,),
        in_specs=[
            pl.BlockSpec((NC, BLK, C), lambda i: (0, i, 0)),
            pl.BlockSpec((BLK, C), lambda i: (i, 0)),
            pl.BlockSpec((BLK, 1), lambda i: (i, 0)),
            pl.BlockSpec((1, C), lambda i: (0, 0)),
            pl.BlockSpec((BLK, 1), lambda i: (i, 0)),
        ],
        out_specs=[
            pl.BlockSpec((BLK, C), lambda i: (i, 0)),
            pl.BlockSpec((G, C), lambda i: (0, 0)),
        ],
        out_shape=[
            jax.ShapeDtypeStruct((NP, C), jnp.float32),
            jax.ShapeDtypeStruct((G, C), jnp.float32),
        ],
    )(accp, hs2, dinv, b2, bat)


@jax.jit
def kernel(x, edge_index, batch, W1, b1, W2, b2):
    padr = N + (jnp.arange(NW * PADE, dtype=jnp.int32)
                % (NP - N)).reshape(NW, PADE)
    src3 = jnp.concatenate(
        [edge_index[0].reshape(NW, EPT), padr], axis=1).reshape(NW, NCH, K)
    dst3 = jnp.concatenate(
        [edge_index[1].reshape(NW, EPT), padr], axis=1).reshape(NW, NCH, K)
    x_p = jnp.pad(x, ((0, NP - N), (0, 0)))
    bat = jnp.pad(batch, (0, NP - N), constant_values=G).reshape(NP, 1)

    degp = _deg_kernel(dst3).reshape(NC, NP, 1)
    hs1, dinv = _call_mm1(degp, x_p, W1)
    accp1 = _prop_kernel(hs1, src3, dst3)
    h1, hs2, sx1 = _call_mid(accp1, hs1, dinv, b1.reshape(1, C), W2, bat)
    accp2 = _prop_kernel(hs2, src3, dst3)
    h2, sx2 = _call_fin(accp2, hs2, dinv, b2.reshape(1, C), bat)
    return h2[:N], jnp.concatenate([sx1, sx2], axis=1)


# speedup vs baseline: 1.1197x; 1.1197x over previous
import functools

import jax
import jax.numpy as jnp
from jax import lax
from jax.experimental import pallas as pl
from jax.experimental.pallas import tpu as pltpu
from jax.experimental.pallas import tpu_sc as plsc

N = 10000
NP = 10240
E = 320000
C = 128
G = 64

NC = 2
NS = 16
NW = NC * NS
EPT = E // NW
K = 80
NCH = EPT // K
RPT = NP // NS
RB = RPT // K

_mesh = plsc.VectorSubcoreMesh(core_axis_name="c", subcore_axis_name="s")


@functools.partial(
    pl.kernel,
    out_type=jax.ShapeDtypeStruct((NC, NP), jnp.float32),
    mesh=_mesh,
    scratch_types=[
        pltpu.VMEM((NCH, K), jnp.int32),
        pltpu.VMEM((K,), jnp.float32),
        pltpu.VMEM((RPT,), jnp.float32),
        pltpu.VMEM_SHARED((NP,), jnp.float32),
        pltpu.SemaphoreType.DMA,
    ],
)
def _deg_kernel(dst_hbm, out_hbm, idx_v, ones_v, stage_v, deg_sh, sem):
    c = lax.axis_index("c")
    s = lax.axis_index("s")
    wid = c * NS + s

    def fill_stage(i, _):
        stage_v[pl.ds(i * 16, 16)] = jnp.zeros((16,), jnp.float32)
        return 0
    lax.fori_loop(0, RPT // 16, fill_stage, 0)

    def fill_ones(i, _):
        ones_v[pl.ds(i * 16, 16)] = jnp.full((16,), 1.0, jnp.float32)
        return 0
    lax.fori_loop(0, K // 16, fill_ones, 0)

    pltpu.sync_copy(stage_v, deg_sh.at[pl.ds(s * RPT, RPT)])
    plsc.subcore_barrier()

    pltpu.sync_copy(dst_hbm.at[wid], idx_v)

    def body(j, _):
        pltpu.async_copy(ones_v, deg_sh.at[idx_v.at[j]], sem,
                         add=True).wait()
        return 0
    lax.fori_loop(0, NCH, body, 0)
    plsc.subcore_barrier()

    pltpu.sync_copy(deg_sh.at[pl.ds(s * RPT, RPT)], stage_v)
    pltpu.sync_copy(stage_v, out_hbm.at[c, pl.ds(s * RPT, RPT)])


@functools.partial(
    pl.kernel,
    out_type=jax.ShapeDtypeStruct((NC, NP, C), jnp.float32),
    mesh=_mesh,
    scratch_types=[
        pltpu.VMEM((64, K), jnp.int32),
        pltpu.VMEM((64, K), jnp.int32),
        pltpu.VMEM((3, K, C), jnp.float32),
        pltpu.VMEM_SHARED((NP, C), jnp.float32),
        pltpu.SemaphoreType.DMA,
        pltpu.SemaphoreType.DMA,
        pltpu.SemaphoreType.DMA,
        pltpu.SemaphoreType.DMA,
        pltpu.SemaphoreType.DMA,
        pltpu.SemaphoreType.DMA,
    ],
)
def _prop_kernel(hs_hbm, src_hbm, dst_hbm, out_hbm,
                 src_v, dst_v, rows_v, acc_sh,
                 gsem0, gsem1, gsem2, ssem0, ssem1, ssem2):
    c = lax.axis_index("c")
    s = lax.axis_index("s")
    wid = c * NS + s
    gsems = (gsem0, gsem1, gsem2)
    ssems = (ssem0, ssem1, ssem2)

    def fill_z(r, _):
        def fill_row(i, _):
            rows_v[0, r, pl.ds(i * 16, 16)] = jnp.zeros((16,), jnp.float32)
            return 0
        lax.fori_loop(0, C // 16, fill_row, 0)
        return 0
    lax.fori_loop(0, K, fill_z, 0)

    def zero_acc(r, _):
        pltpu.sync_copy(rows_v.at[0],
                        acc_sh.at[pl.ds(s * RPT + r * K, K)])
        return 0
    lax.fori_loop(0, RB, zero_acc, 0)
    plsc.subcore_barrier()

    def run_phase(base, nch):
        pltpu.sync_copy(src_hbm.at[wid, pl.ds(base, nch)],
                        src_v.at[pl.ds(0, nch)])
        pltpu.sync_copy(dst_hbm.at[wid, pl.ds(base, nch)],
                        dst_v.at[pl.ds(0, nch)])

        def gather(j, slot):
            return pltpu.async_copy(
                hs_hbm.at[src_v.at[j]], rows_v.at[slot], gsems[slot])

        gather(0, 0)
        gather(1, 1)

        def body(j, _):
            for r in range(3):
                rp2 = (r + 2) % 3

                @pl.when(lax.rem(j, 3) == r)
                def _():
                    @pl.when(j >= 1)
                    def _():
                        pltpu.make_async_copy(
                            rows_v.at[rp2],
                            acc_sh.at[dst_v.at[j - 1]],
                            ssems[rp2]).wait()

                    @pl.when(j + 2 < nch)
                    def _():
                        gather(j + 2, rp2)
                    pltpu.make_async_copy(
                        hs_hbm.at[src_v.at[j]],
                        rows_v.at[r], gsems[r]).wait()
                    pltpu.async_copy(rows_v.at[r],
                                     acc_sh.at[dst_v.at[j]], ssems[r],
                                     add=True)
            return 0
        lax.fori_loop(0, nch, body, 0)
        t = nch - 1
        pltpu.make_async_copy(rows_v.at[t % 3],
                              acc_sh.at[dst_v.at[t]],
                              ssems[t % 3]).wait()

    run_phase(0, 64)
    run_phase(64, NCH - 64)
    plsc.subcore_barrier()

    def wb(r, _):
        slot = rows_v.at[0]
        pltpu.sync_copy(acc_sh.at[pl.ds(s * RPT + r * K, K)], slot)
        pltpu.sync_copy(slot, out_hbm.at[c, pl.ds(s * RPT + r * K, K)])
        return 0
    lax.fori_loop(0, RB, wb, 0)


BLK = 1024


def _mm1_body(degp_ref, x_ref, w_ref, hs_ref, dinv_ref):
    deg = degp_ref[0] + degp_ref[1] + 1.0
    dinv = lax.rsqrt(deg)
    h = jnp.dot(x_ref[...], w_ref[...], preferred_element_type=jnp.float32)
    hs_ref[...] = dinv * h
    dinv_ref[...] = dinv


def _call_mm1(degp, x_p, W1):
    return pl.pallas_call(
        _mm1_body,
        grid=(NP // BLK,),
        in_specs=[
            pl.BlockSpec((NC, BLK, 1), lambda i: (0, i, 0)),
            pl.BlockSpec((BLK, C), lambda i: (i, 0)),
            pl.BlockSpec((C, C), lambda i: (0, 0)),
        ],
        out_specs=[
            pl.BlockSpec((BLK, C), lambda i: (i, 0)),
            pl.BlockSpec((BLK, 1), lambda i: (i, 0)),
        ],
        out_shape=[
            jax.ShapeDtypeStruct((NP, C), jnp.float32),
            jax.ShapeDtypeStruct((NP, 1), jnp.float32),
        ],
    )(degp, x_p, W1)


def _mid_body(accp_ref, hs_ref, dinv_ref, b_ref, w_ref, bat_ref,
              h1_ref, hs2_ref, sx_ref):
    i = pl.program_id(0)
    dinv = dinv_ref[...]
    pre = dinv * (accp_ref[0] + accp_ref[1] + hs_ref[...]) + b_ref[...]
    h1 = jnp.maximum(pre, 0.0)
    h1_ref[...] = h1
    hs2_ref[...] = dinv * jnp.dot(h1, w_ref[...],
                                  preferred_element_type=jnp.float32)
    oh = (bat_ref[...] == lax.broadcasted_iota(jnp.int32, (1, G), 1))
    oh = oh.astype(jnp.float32)
    part = lax.dot_general(oh, h1, (((0,), (0,)), ((), ())),
                           preferred_element_type=jnp.float32)

    @pl.when(i == 0)
    def _():
        sx_ref[...] = jnp.zeros_like(sx_ref)
    sx_ref[...] += part


def _call_mid(accp, hs1, dinv, b1, W2, bat):
    return pl.pallas_call(
        _mid_body,
        grid=(NP // BLK,),
        in_specs=[
            pl.BlockSpec((NC, BLK, C), lambda i: (0, i, 0)),
            pl.BlockSpec((BLK, C), lambda i: (i, 0)),
            pl.BlockSpec((BLK, 1), lambda i: (i, 0)),
            pl.BlockSpec((1, C), lambda i: (0, 0)),
            pl.BlockSpec((C, C), lambda i: (0, 0)),
            pl.BlockSpec((BLK, 1), lambda i: (i, 0)),
        ],
        out_specs=[
            pl.BlockSpec((BLK, C), lambda i: (i, 0)),
            pl.BlockSpec((BLK, C), lambda i: (i, 0)),
            pl.BlockSpec((G, C), lambda i: (0, 0)),
        ],
        out_shape=[
            jax.ShapeDtypeStruct((NP, C), jnp.float32),
            jax.ShapeDtypeStruct((NP, C), jnp.float32),
            jax.ShapeDtypeStruct((G, C), jnp.float32),
        ],
    )(accp, hs1, dinv, b1, W2, bat)


def _fin_body(accp_ref, hs_ref, dinv_ref, b_ref, bat_ref, h2_ref, sx_ref):
    i = pl.program_id(0)
    dinv = dinv_ref[...]
    pre = dinv * (accp_ref[0] + accp_ref[1] + hs_ref[...]) + b_ref[...]
    h2 = jnp.maximum(pre, 0.0)
    h2_ref[...] = h2
    oh = (bat_ref[...] == lax.broadcasted_iota(jnp.int32, (1, G), 1))
    oh = oh.astype(jnp.float32)
    part = lax.dot_general(oh, h2, (((0,), (0,)), ((), ())),
                           preferred_element_type=jnp.float32)

    @pl.when(i == 0)
    def _():
        sx_ref[...] = jnp.zeros_like(sx_ref)
    sx_ref[...] += part


def _call_fin(accp, hs2, dinv, b2, bat):
    return pl.pallas_call(
        _fin_body,
        grid=(NP // BLK,),
        in_specs=[
            pl.BlockSpec((NC, BLK, C), lambda i: (0, i, 0)),
            pl.BlockSpec((BLK, C), lambda i: (i, 0)),
            pl.BlockSpec((BLK, 1), lambda i: (i, 0)),
            pl.BlockSpec((1, C), lambda i: (0, 0)),
            pl.BlockSpec((BLK, 1), lambda i: (i, 0)),
        ],
        out_specs=[
            pl.BlockSpec((BLK, C), lambda i: (i, 0)),
            pl.BlockSpec((G, C), lambda i: (0, 0)),
        ],
        out_shape=[
            jax.ShapeDtypeStruct((NP, C), jnp.float32),
            jax.ShapeDtypeStruct((G, C), jnp.float32),
        ],
    )(accp, hs2, dinv, b2, bat)


@jax.jit
def kernel(x, edge_index, batch, W1, b1, W2, b2):
    src3 = edge_index[0].reshape(NW, NCH, K)
    dst3 = edge_index[1].reshape(NW, NCH, K)
    x_p = jnp.pad(x, ((0, NP - N), (0, 0)))
    bat = jnp.pad(batch, (0, NP - N), constant_values=G).reshape(NP, 1)

    degp = _deg_kernel(dst3).reshape(NC, NP, 1)
    hs1, dinv = _call_mm1(degp, x_p, W1)
    accp1 = _prop_kernel(hs1, src3, dst3)
    h1, hs2, sx1 = _call_mid(accp1, hs1, dinv, b1.reshape(1, C), W2, bat)
    accp2 = _prop_kernel(hs2, src3, dst3)
    h2, sx2 = _call_fin(accp2, hs2, dinv, b2.reshape(1, C), bat)
    return h2[:N], jnp.concatenate([sx1, sx2], axis=1)


# speedup vs baseline: 1.1368x; 1.0153x over previous
import functools

import jax
import jax.numpy as jnp
from jax import lax
from jax.experimental import pallas as pl
from jax.experimental.pallas import tpu as pltpu
from jax.experimental.pallas import tpu_sc as plsc

N = 10000
NP = 10240
E = 320000
C = 128
G = 64

NC = 2
NS = 16
NW = NC * NS
EPT = E // NW
K = 80
NCH = EPT // K
RPT = NP // NS
RB = RPT // K

_mesh = plsc.VectorSubcoreMesh(core_axis_name="c", subcore_axis_name="s")


@functools.partial(
    pl.kernel,
    out_type=jax.ShapeDtypeStruct((NC, NP), jnp.float32),
    mesh=_mesh,
    scratch_types=[
        pltpu.VMEM((NCH, K), jnp.int32),
        pltpu.VMEM((K,), jnp.float32),
        pltpu.VMEM((RPT,), jnp.float32),
        pltpu.VMEM_SHARED((NP,), jnp.float32),
        pltpu.SemaphoreType.DMA,
    ],
)
def _deg_kernel(dst_hbm, out_hbm, idx_v, ones_v, stage_v, deg_sh, sem):
    c = lax.axis_index("c")
    s = lax.axis_index("s")
    wid = c * NS + s

    def fill_stage(i, _):
        stage_v[pl.ds(i * 16, 16)] = jnp.zeros((16,), jnp.float32)
        return 0
    lax.fori_loop(0, RPT // 16, fill_stage, 0)

    def fill_ones(i, _):
        ones_v[pl.ds(i * 16, 16)] = jnp.full((16,), 1.0, jnp.float32)
        return 0
    lax.fori_loop(0, K // 16, fill_ones, 0)

    pltpu.sync_copy(stage_v, deg_sh.at[pl.ds(s * RPT, RPT)])
    plsc.subcore_barrier()

    pltpu.sync_copy(dst_hbm.at[wid], idx_v)

    def body(j, _):
        pltpu.async_copy(ones_v, deg_sh.at[idx_v.at[j]], sem,
                         add=True).wait()
        return 0
    lax.fori_loop(0, NCH, body, 0)
    plsc.subcore_barrier()

    pltpu.sync_copy(deg_sh.at[pl.ds(s * RPT, RPT)], stage_v)
    pltpu.sync_copy(stage_v, out_hbm.at[c, pl.ds(s * RPT, RPT)])


@functools.partial(
    pl.kernel,
    out_type=jax.ShapeDtypeStruct((NC, NP, C), jnp.float32),
    mesh=_mesh,
    scratch_types=[
        pltpu.VMEM((64, K), jnp.int32),
        pltpu.VMEM((64, K), jnp.int32),
        pltpu.VMEM((3, K, C), jnp.float32),
        pltpu.VMEM_SHARED((NP, C), jnp.float32),
        pltpu.SemaphoreType.DMA,
        pltpu.SemaphoreType.DMA,
        pltpu.SemaphoreType.DMA,
        pltpu.SemaphoreType.DMA,
        pltpu.SemaphoreType.DMA,
        pltpu.SemaphoreType.DMA,
    ],
)
def _prop_kernel(hs_hbm, src_hbm, dst_hbm, out_hbm,
                 src_v, dst_v, rows_v, acc_sh,
                 gsem0, gsem1, gsem2, ssem0, ssem1, ssem2):
    c = lax.axis_index("c")
    s = lax.axis_index("s")
    wid = c * NS + s
    gsems = (gsem0, gsem1, gsem2)
    ssems = (ssem0, ssem1, ssem2)

    def fill_z(r, _):
        def fill_row(i, _):
            rows_v[0, r, pl.ds(i * 16, 16)] = jnp.zeros((16,), jnp.float32)
            return 0
        lax.fori_loop(0, C // 16, fill_row, 0)
        return 0
    lax.fori_loop(0, K, fill_z, 0)

    def zero_acc(r, _):
        pltpu.sync_copy(rows_v.at[0],
                        acc_sh.at[pl.ds(s * RPT + r * K, K)])
        return 0
    lax.fori_loop(0, RB, zero_acc, 0)
    plsc.subcore_barrier()

    def run_phase(base, nch):
        pltpu.sync_copy(src_hbm.at[wid, pl.ds(base, nch)],
                        src_v.at[pl.ds(0, nch)])
        pltpu.sync_copy(dst_hbm.at[wid, pl.ds(base, nch)],
                        dst_v.at[pl.ds(0, nch)])

        def gather(j, slot):
            return pltpu.async_copy(
                hs_hbm.at[src_v.at[j]], rows_v.at[slot], gsems[slot])

        gather(0, 0)
        gather(1, 1)

        def body(j, _):
            for r in range(3):
                rp2 = (r + 2) % 3

                @pl.when(lax.rem(j, 3) == r)
                def _():
                    @pl.when(j >= 1)
                    def _():
                        pltpu.make_async_copy(
                            rows_v.at[rp2],
                            acc_sh.at[dst_v.at[j - 1]],
                            ssems[rp2]).wait()

                    @pl.when(j + 2 < nch)
                    def _():
                        gather(j + 2, rp2)
                    pltpu.make_async_copy(
                        hs_hbm.at[src_v.at[j]],
                        rows_v.at[r], gsems[r]).wait()
                    pltpu.async_copy(rows_v.at[r],
                                     acc_sh.at[dst_v.at[j]], ssems[r],
                                     add=True)
            return 0
        lax.fori_loop(0, nch, body, 0)
        t = nch - 1
        pltpu.make_async_copy(rows_v.at[t % 3],
                              acc_sh.at[dst_v.at[t]],
                              ssems[t % 3]).wait()

    run_phase(0, 64)
    run_phase(64, NCH - 64)
    plsc.subcore_barrier()

    def wb(r, _):
        slot = rows_v.at[0]
        pltpu.sync_copy(acc_sh.at[pl.ds(s * RPT + r * K, K)], slot)
        pltpu.sync_copy(slot, out_hbm.at[c, pl.ds(s * RPT + r * K, K)])
        return 0
    lax.fori_loop(0, RB, wb, 0)


BLK = 2048


def _mm1_body(degp_ref, x_ref, w_ref, hs_ref, dinv_ref):
    deg = degp_ref[0] + degp_ref[1] + 1.0
    dinv = lax.rsqrt(deg)
    h = jnp.dot(x_ref[...], w_ref[...], preferred_element_type=jnp.float32)
    hs_ref[...] = dinv * h
    dinv_ref[...] = dinv


def _call_mm1(degp, x_p, W1):
    return pl.pallas_call(
        _mm1_body,
        grid=(NP // BLK,),
        in_specs=[
            pl.BlockSpec((NC, BLK, 1), lambda i: (0, i, 0)),
            pl.BlockSpec((BLK, C), lambda i: (i, 0)),
            pl.BlockSpec((C, C), lambda i: (0, 0)),
        ],
        out_specs=[
            pl.BlockSpec((BLK, C), lambda i: (i, 0)),
            pl.BlockSpec((BLK, 1), lambda i: (i, 0)),
        ],
        out_shape=[
            jax.ShapeDtypeStruct((NP, C), jnp.float32),
            jax.ShapeDtypeStruct((NP, 1), jnp.float32),
        ],
    )(degp, x_p, W1)


def _mid_body(accp_ref, hs_ref, dinv_ref, b_ref, w_ref, bat_ref,
              h1_ref, hs2_ref, sx_ref):
    i = pl.program_id(0)
    dinv = dinv_ref[...]
    pre = dinv * (accp_ref[0] + accp_ref[1] + hs_ref[...]) + b_ref[...]
    h1 = jnp.maximum(pre, 0.0)
    h1_ref[...] = h1
    hs2_ref[...] = dinv * jnp.dot(h1, w_ref[...],
                                  preferred_element_type=jnp.float32)
    oh = (bat_ref[...] == lax.broadcasted_iota(jnp.int32, (1, G), 1))
    oh = oh.astype(jnp.float32)
    part = lax.dot_general(oh, h1, (((0,), (0,)), ((), ())),
                           preferred_element_type=jnp.float32)

    @pl.when(i == 0)
    def _():
        sx_ref[...] = jnp.zeros_like(sx_ref)
    sx_ref[...] += part


def _call_mid(accp, hs1, dinv, b1, W2, bat):
    return pl.pallas_call(
        _mid_body,
        grid=(NP // BLK,),
        in_specs=[
            pl.BlockSpec((NC, BLK, C), lambda i: (0, i, 0)),
            pl.BlockSpec((BLK, C), lambda i: (i, 0)),
            pl.BlockSpec((BLK, 1), lambda i: (i, 0)),
            pl.BlockSpec((1, C), lambda i: (0, 0)),
            pl.BlockSpec((C, C), lambda i: (0, 0)),
            pl.BlockSpec((BLK, 1), lambda i: (i, 0)),
        ],
        out_specs=[
            pl.BlockSpec((BLK, C), lambda i: (i, 0)),
            pl.BlockSpec((BLK, C), lambda i: (i, 0)),
            pl.BlockSpec((G, C), lambda i: (0, 0)),
        ],
        out_shape=[
            jax.ShapeDtypeStruct((NP, C), jnp.float32),
            jax.ShapeDtypeStruct((NP, C), jnp.float32),
            jax.ShapeDtypeStruct((G, C), jnp.float32),
        ],
    )(accp, hs1, dinv, b1, W2, bat)


def _fin_body(accp_ref, hs_ref, dinv_ref, b_ref, bat_ref, h2_ref, sx_ref):
    i = pl.program_id(0)
    dinv = dinv_ref[...]
    pre = dinv * (accp_ref[0] + accp_ref[1] + hs_ref[...]) + b_ref[...]
    h2 = jnp.maximum(pre, 0.0)
    h2_ref[...] = h2
    oh = (bat_ref[...] == lax.broadcasted_iota(jnp.int32, (1, G), 1))
    oh = oh.astype(jnp.float32)
    part = lax.dot_general(oh, h2, (((0,), (0,)), ((), ())),
                           preferred_element_type=jnp.float32)

    @pl.when(i == 0)
    def _():
        sx_ref[...] = jnp.zeros_like(sx_ref)
    sx_ref[...] += part


def _call_fin(accp, hs2, dinv, b2, bat):
    return pl.pallas_call(
        _fin_body,
        grid=(NP // BLK,),
        in_specs=[
            pl.BlockSpec((NC, BLK, C), lambda i: (0, i, 0)),
            pl.BlockSpec((BLK, C), lambda i: (i, 0)),
            pl.BlockSpec((BLK, 1), lambda i: (i, 0)),
            pl.BlockSpec((1, C), lambda i: (0, 0)),
            pl.BlockSpec((BLK, 1), lambda i: (i, 0)),
        ],
        out_specs=[
            pl.BlockSpec((BLK, C), lambda i: (i, 0)),
            pl.BlockSpec((G, C), lambda i: (0, 0)),
        ],
        out_shape=[
            jax.ShapeDtypeStruct((NP, C), jnp.float32),
            jax.ShapeDtypeStruct((G, C), jnp.float32),
        ],
    )(accp, hs2, dinv, b2, bat)


@jax.jit
def kernel(x, edge_index, batch, W1, b1, W2, b2):
    src3 = edge_index[0].reshape(NW, NCH, K)
    dst3 = edge_index[1].reshape(NW, NCH, K)
    x_p = jnp.pad(x, ((0, NP - N), (0, 0)))
    bat = jnp.pad(batch, (0, NP - N), constant_values=G).reshape(NP, 1)

    degp = _deg_kernel(dst3).reshape(NC, NP, 1)
    hs1, dinv = _call_mm1(degp, x_p, W1)
    accp1 = _prop_kernel(hs1, src3, dst3)
    h1, hs2, sx1 = _call_mid(accp1, hs1, dinv, b1.reshape(1, C), W2, bat)
    accp2 = _prop_kernel(hs2, src3, dst3)
    h2, sx2 = _call_fin(accp2, hs2, dinv, b2.reshape(1, C), bat)
    return h2[:N], jnp.concatenate([sx1, sx2], axis=1)


# speedup vs baseline: 1.1388x; 1.0017x over previous
import functools

import jax
import jax.numpy as jnp
from jax import lax
from jax.experimental import pallas as pl
from jax.experimental.pallas import tpu as pltpu
from jax.experimental.pallas import tpu_sc as plsc

N = 10000
NP = 10240
E = 320000
C = 128
G = 64

NC = 2
NS = 16
NW = NC * NS
EPT = E // NW
K = 80
NCH = EPT // K
RPT = NP // NS
RB = RPT // K

_mesh = plsc.VectorSubcoreMesh(core_axis_name="c", subcore_axis_name="s")


@functools.partial(
    pl.kernel,
    out_type=jax.ShapeDtypeStruct((NC, NP), jnp.float32),
    mesh=_mesh,
    scratch_types=[
        pltpu.VMEM((NCH, K), jnp.int32),
        pltpu.VMEM((K,), jnp.float32),
        pltpu.VMEM((RPT,), jnp.float32),
        pltpu.VMEM_SHARED((NP,), jnp.float32),
        pltpu.SemaphoreType.DMA,
    ],
)
def _deg_kernel(dst_hbm, out_hbm, idx_v, ones_v, stage_v, deg_sh, sem):
    c = lax.axis_index("c")
    s = lax.axis_index("s")
    wid = c * NS + s

    def fill_stage(i, _):
        stage_v[pl.ds(i * 16, 16)] = jnp.zeros((16,), jnp.float32)
        return 0
    lax.fori_loop(0, RPT // 16, fill_stage, 0)

    def fill_ones(i, _):
        ones_v[pl.ds(i * 16, 16)] = jnp.full((16,), 1.0, jnp.float32)
        return 0
    lax.fori_loop(0, K // 16, fill_ones, 0)

    pltpu.sync_copy(stage_v, deg_sh.at[pl.ds(s * RPT, RPT)])
    plsc.subcore_barrier()

    pltpu.sync_copy(dst_hbm.at[wid], idx_v)

    def body(j, _):
        pltpu.async_copy(ones_v, deg_sh.at[idx_v.at[j]], sem,
                         add=True).wait()
        return 0
    lax.fori_loop(0, NCH, body, 0)
    plsc.subcore_barrier()

    pltpu.sync_copy(deg_sh.at[pl.ds(s * RPT, RPT)], stage_v)
    pltpu.sync_copy(stage_v, out_hbm.at[c, pl.ds(s * RPT, RPT)])


@functools.partial(
    pl.kernel,
    out_type=jax.ShapeDtypeStruct((NC, NP, C), jnp.float32),
    mesh=_mesh,
    scratch_types=[
        pltpu.VMEM((64, K), jnp.int32),
        pltpu.VMEM((64, K), jnp.int32),
        pltpu.VMEM((3, K, C), jnp.float32),
        pltpu.VMEM_SHARED((NP, C), jnp.float32),
        pltpu.SemaphoreType.DMA,
        pltpu.SemaphoreType.DMA,
        pltpu.SemaphoreType.DMA,
        pltpu.SemaphoreType.DMA,
        pltpu.SemaphoreType.DMA,
        pltpu.SemaphoreType.DMA,
    ],
)
def _prop_kernel(hs_hbm, src_hbm, dst_hbm, out_hbm,
                 src_v, dst_v, rows_v, acc_sh,
                 gsem0, gsem1, gsem2, ssem0, ssem1, ssem2):
    c = lax.axis_index("c")
    s = lax.axis_index("s")
    wid = c * NS + s
    gsems = (gsem0, gsem1, gsem2)
    ssems = (ssem0, ssem1, ssem2)

    def fill_z(r, _):
        def fill_row(i, _):
            rows_v[0, r, pl.ds(i * 16, 16)] = jnp.zeros((16,), jnp.float32)
            return 0
        lax.fori_loop(0, C // 16, fill_row, 0)
        return 0
    lax.fori_loop(0, K, fill_z, 0)

    def zero_acc(r, _):
        pltpu.sync_copy(rows_v.at[0],
                        acc_sh.at[pl.ds(s * RPT + r * K, K)])
        return 0
    lax.fori_loop(0, RB, zero_acc, 0)
    plsc.subcore_barrier()

    def run_phase(base, nch):
        pltpu.sync_copy(src_hbm.at[wid, pl.ds(base, nch)],
                        src_v.at[pl.ds(0, nch)])
        pltpu.sync_copy(dst_hbm.at[wid, pl.ds(base, nch)],
                        dst_v.at[pl.ds(0, nch)])

        def gather(j, slot):
            return pltpu.async_copy(
                hs_hbm.at[src_v.at[j]], rows_v.at[slot], gsems[slot])

        gather(0, 0)
        gather(1, 1)

        def body(j, _):
            for r in range(3):
                rp2 = (r + 2) % 3

                @pl.when(lax.rem(j, 3) == r)
                def _():
                    @pl.when(j >= 1)
                    def _():
                        pltpu.make_async_copy(
                            rows_v.at[rp2],
                            acc_sh.at[dst_v.at[j - 1]],
                            ssems[rp2]).wait()

                    @pl.when(j + 2 < nch)
                    def _():
                        gather(j + 2, rp2)
                    pltpu.make_async_copy(
                        hs_hbm.at[src_v.at[j]],
                        rows_v.at[r], gsems[r]).wait()
                    pltpu.async_copy(rows_v.at[r],
                                     acc_sh.at[dst_v.at[j]], ssems[r],
                                     add=True)
            return 0
        lax.fori_loop(0, nch, body, 0)
        t = nch - 1
        pltpu.make_async_copy(rows_v.at[t % 3],
                              acc_sh.at[dst_v.at[t]],
                              ssems[t % 3]).wait()

    run_phase(0, 64)
    run_phase(64, NCH - 64)
    plsc.subcore_barrier()

    def wb(r, _):
        slot = rows_v.at[0]
        pltpu.sync_copy(acc_sh.at[pl.ds(s * RPT + r * K, K)], slot)
        pltpu.sync_copy(slot, out_hbm.at[c, pl.ds(s * RPT + r * K, K)])
        return 0
    lax.fori_loop(0, RB, wb, 0)


BLK = NP


def _mm1_body(degp_ref, x_ref, w_ref, hs_ref, dinv_ref):
    deg = degp_ref[0] + degp_ref[1] + 1.0
    dinv = lax.rsqrt(deg)
    h = jnp.dot(x_ref[...], w_ref[...], preferred_element_type=jnp.float32)
    hs_ref[...] = dinv * h
    dinv_ref[...] = dinv


def _call_mm1(degp, x_p, W1):
    return pl.pallas_call(
        _mm1_body,
        grid=(NP // BLK,),
        in_specs=[
            pl.BlockSpec((NC, BLK, 1), lambda i: (0, i, 0)),
            pl.BlockSpec((BLK, C), lambda i: (i, 0)),
            pl.BlockSpec((C, C), lambda i: (0, 0)),
        ],
        out_specs=[
            pl.BlockSpec((BLK, C), lambda i: (i, 0)),
            pl.BlockSpec((BLK, 1), lambda i: (i, 0)),
        ],
        out_shape=[
            jax.ShapeDtypeStruct((NP, C), jnp.float32),
            jax.ShapeDtypeStruct((NP, 1), jnp.float32),
        ],
    )(degp, x_p, W1)


def _mid_body(accp_ref, hs_ref, dinv_ref, b_ref, w_ref, bat_ref,
              h1_ref, hs2_ref, sx_ref):
    i = pl.program_id(0)
    dinv = dinv_ref[...]
    pre = dinv * (accp_ref[0] + accp_ref[1] + hs_ref[...]) + b_ref[...]
    h1 = jnp.maximum(pre, 0.0)
    h1_ref[...] = h1
    hs2_ref[...] = dinv * jnp.dot(h1, w_ref[...],
                                  preferred_element_type=jnp.float32)
    oh = (bat_ref[...] == lax.broadcasted_iota(jnp.int32, (1, G), 1))
    oh = oh.astype(jnp.float32)
    part = lax.dot_general(oh, h1, (((0,), (0,)), ((), ())),
                           preferred_element_type=jnp.float32)

    @pl.when(i == 0)
    def _():
        sx_ref[...] = jnp.zeros_like(sx_ref)
    sx_ref[...] += part


def _call_mid(accp, hs1, dinv, b1, W2, bat):
    return pl.pallas_call(
        _mid_body,
        grid=(NP // BLK,),
        in_specs=[
            pl.BlockSpec((NC, BLK, C), lambda i: (0, i, 0)),
            pl.BlockSpec((BLK, C), lambda i: (i, 0)),
            pl.BlockSpec((BLK, 1), lambda i: (i, 0)),
            pl.BlockSpec((1, C), lambda i: (0, 0)),
            pl.BlockSpec((C, C), lambda i: (0, 0)),
            pl.BlockSpec((BLK, 1), lambda i: (i, 0)),
        ],
        out_specs=[
            pl.BlockSpec((BLK, C), lambda i: (i, 0)),
            pl.BlockSpec((BLK, C), lambda i: (i, 0)),
            pl.BlockSpec((G, C), lambda i: (0, 0)),
        ],
        out_shape=[
            jax.ShapeDtypeStruct((NP, C), jnp.float32),
            jax.ShapeDtypeStruct((NP, C), jnp.float32),
            jax.ShapeDtypeStruct((G, C), jnp.float32),
        ],
    )(accp, hs1, dinv, b1, W2, bat)


def _fin_body(accp_ref, hs_ref, dinv_ref, b_ref, bat_ref, h2_ref, sx_ref):
    i = pl.program_id(0)
    dinv = dinv_ref[...]
    pre = dinv * (accp_ref[0] + accp_ref[1] + hs_ref[...]) + b_ref[...]
    h2 = jnp.maximum(pre, 0.0)
    h2_ref[...] = h2
    oh = (bat_ref[...] == lax.broadcasted_iota(jnp.int32, (1, G), 1))
    oh = oh.astype(jnp.float32)
    part = lax.dot_general(oh, h2, (((0,), (0,)), ((), ())),
                           preferred_element_type=jnp.float32)

    @pl.when(i == 0)
    def _():
        sx_ref[...] = jnp.zeros_like(sx_ref)
    sx_ref[...] += part


def _call_fin(accp, hs2, dinv, b2, bat):
    return pl.pallas_call(
        _fin_body,
        grid=(NP // BLK,),
        in_specs=[
            pl.BlockSpec((NC, BLK, C), lambda i: (0, i, 0)),
            pl.BlockSpec((BLK, C), lambda i: (i, 0)),
            pl.BlockSpec((BLK, 1), lambda i: (i, 0)),
            pl.BlockSpec((1, C), lambda i: (0, 0)),
            pl.BlockSpec((BLK, 1), lambda i: (i, 0)),
        ],
        out_specs=[
            pl.BlockSpec((BLK, C), lambda i: (i, 0)),
            pl.BlockSpec((G, C), lambda i: (0, 0)),
        ],
        out_shape=[
            jax.ShapeDtypeStruct((NP, C), jnp.float32),
            jax.ShapeDtypeStruct((G, C), jnp.float32),
        ],
    )(accp, hs2, dinv, b2, bat)


@jax.jit
def kernel(x, edge_index, batch, W1, b1, W2, b2):
    src3 = edge_index[0].reshape(NW, NCH, K)
    dst3 = edge_index[1].reshape(NW, NCH, K)
    x_p = jnp.pad(x, ((0, NP - N), (0, 0)))
    bat = jnp.pad(batch, (0, NP - N), constant_values=G).reshape(NP, 1)

    degp = _deg_kernel(dst3).reshape(NC, NP, 1)
    hs1, dinv = _call_mm1(degp, x_p, W1)
    accp1 = _prop_kernel(hs1, src3, dst3)
    h1, hs2, sx1 = _call_mid(accp1, hs1, dinv, b1.reshape(1, C), W2, bat)
    accp2 = _prop_kernel(hs2, src3, dst3)
    h2, sx2 = _call_fin(accp2, hs2, dinv, b2.reshape(1, C), bat)
    return h2[:N], jnp.concatenate([sx1, sx2], axis=1)
